# baseline (speedup 1.0000x reference)
import jax
import jax.numpy as jnp
from jax import lax
from jax.experimental import pallas as pl
from jax.experimental.pallas import tpu as pltpu
from jax.experimental.pallas import tpu_sc as plsc

_EMBED = 128
_MAX_LEN = 200
_NUM_ITEMS = 100000
_BATCH = 1024

_NC, _NS = 2, 16
_NW = _NC * _NS
_ROWS = _BATCH * _MAX_LEN
_CHUNK = 128
_NCHUNK = _ROWS // _CHUNK
_CPW = _NCHUNK // _NW
_NV = _EMBED // 16
_PE2 = _MAX_LEN + _CHUNK
_NB = 5


def _body(ridx_hbm, sidx_hbm, t0_hbm, tm_hbm, pe_hbm, emb_hbm, out_hbm,
          ridx_v, sidx_v, pe2_sh, t0_v, tm_v, e0_v, eN_v, ob,
          gsem0, gsem1, gsem2, gsem3, gsem4,
          osem0, osem1, osem2, osem3, osem4,
          psem0, psem1, psem2, psem3, psem4):
    gsems = (gsem0, gsem1, gsem2, gsem3, gsem4)
    osems = (osem0, osem1, osem2, osem3, osem4)
    psems = (psem0, psem1, psem2, psem3, psem4)
    sid = lax.axis_index("s")
    w = sid * _NC + lax.axis_index("c")
    base = w * _CPW
    pltpu.sync_copy(ridx_hbm.at[pl.ds(base, _CPW)], ridx_v)
    pltpu.sync_copy(sidx_hbm.at[pl.ds(base, _CPW)], sidx_v)
    pltpu.sync_copy(t0_hbm, t0_v)
    pltpu.sync_copy(tm_hbm, tm_v)
    pltpu.sync_copy(emb_hbm.at[pl.ds(0, 1)], e0_v)
    pltpu.sync_copy(emb_hbm.at[pl.ds(_NUM_ITEMS - 1, 1)], eN_v)

    @pl.when(sid == 0)
    def _stage_pe_shared():
        bounce = ob.at[_NB - 1]
        pltpu.sync_copy(pe_hbm.at[pl.ds(0, _CHUNK)], bounce)
        pltpu.sync_copy(bounce, pe2_sh.at[pl.ds(0, _CHUNK)])
        pltpu.sync_copy(bounce, pe2_sh.at[pl.ds(_MAX_LEN, _CHUNK)])
        rest = _MAX_LEN - _CHUNK
        pltpu.sync_copy(pe_hbm.at[pl.ds(_CHUNK, rest)],
                        bounce.at[pl.ds(0, rest)])
        pltpu.sync_copy(bounce.at[pl.ds(0, rest)],
                        pe2_sh.at[pl.ds(_CHUNK, rest)])

    plsc.subcore_barrier()

    d0r = [t0_v[0, pl.ds(16 * j, 16)] - e0_v[0, pl.ds(16 * j, 16)]
           for j in range(_NV)]
    dmr = [tm_v[0, pl.ds(16 * j, 16)] - eN_v[0, pl.ds(16 * j, 16)]
           for j in range(_NV)]
    zf = jnp.zeros((16,), jnp.float32)

    def start_prefill(c, b):
        tc = (c * _CHUNK) % _MAX_LEN
        pltpu.async_copy(pe2_sh.at[pl.ds(tc, _CHUNK)], ob.at[b], psems[b])

    def wait_prefill(b):
        pltpu.make_async_copy(pe2_sh.at[pl.ds(0, _CHUNK)], ob.at[b],
                              psems[b]).wait()

    def start_gather(c, b):
        pltpu.async_copy(emb_hbm.at[sidx_v.at[c]], ob.at[b], gsems[b],
                         add=True)

    def fixup(c, b):
        obb = ob.at[b]
        m = None
        for k in range(_NV):
            iv = ridx_v[c, pl.ds(16 * k, 16)]
            mk = (iv == 0) | (iv == _NUM_ITEMS + 1)
            m = mk if m is None else (m | mk)
        any_spec = jnp.any(m)

        @pl.when(any_spec)
        def _slow():
            @pl.loop(0, _CHUNK)
            def _row(l):
                iv = plsc.load_gather(
                    ridx_v,
                    [jnp.full((16,), c, jnp.int32), jnp.full((16,), l, jnp.int32)])
                f0 = iv == 0
                fm = iv == _NUM_ITEMS + 1

                @pl.when(jnp.any(f0 | fm))
                def _patch():
                    for j in range(_NV):
                        v = obb[l, pl.ds(16 * j, 16)]
                        v = v + jnp.where(f0, d0r[j], jnp.where(fm, dmr[j], zf))
                        obb[l, pl.ds(16 * j, 16)] = v

    def out_slice(c):
        return out_hbm.at[pl.ds((base + c) * _CHUNK, _CHUNK)]

    for j in range(_NB - 1):
        start_prefill(j, j)
    for j in range(_NB - 2):
        wait_prefill(j)
        start_gather(j, j)

    @pl.loop(0, _CPW, step=_NB)
    def _ring(c0):
        for j in range(_NB):
            c = c0 + j
            pltpu.make_async_copy(emb_hbm.at[sidx_v.at[c]], ob.at[j],
                                  gsems[j]).wait()
            fixup(c, j)
            pltpu.async_copy(ob.at[j], out_slice(c), osems[j])

            d4 = c + _NB - 1
            bd4 = (j + _NB - 1) % _NB

            @pl.when(d4 < _CPW)
            def _issue_prefill():
                @pl.when(c >= 1)
                def _wait_prev_out():
                    pltpu.make_async_copy(ob.at[bd4], out_slice(c - 1),
                                          osems[bd4]).wait()

                start_prefill(d4, bd4)

            d3 = c + _NB - 2
            bd3 = (j + _NB - 2) % _NB

            @pl.when(d3 < _CPW)
            def _issue_gather():
                wait_prefill(bd3)
                start_gather(d3, bd3)

    for j in range(_NB):
        c = _CPW - _NB + j
        pltpu.make_async_copy(ob.at[j], out_slice(c), osems[j]).wait()


def kernel(sequence, token_0, token_mask, pe_weight, embeddings):
    emb_out = jnp.copy(embeddings)
    seq = sequence.reshape(_NCHUNK, _CHUNK).astype(jnp.int32)
    sidx = jnp.clip(seq - 1, 0, _NUM_ITEMS - 1)
    mesh = plsc.VectorSubcoreMesh(core_axis_name="c", subcore_axis_name="s",
                                  num_cores=_NC, num_subcores=_NS)
    out = pl.kernel(
        _body,
        out_type=jax.ShapeDtypeStruct((_ROWS, _EMBED), jnp.float32),
        mesh=mesh,
        scratch_types=[
            pltpu.VMEM((_CPW, _CHUNK), jnp.int32),
            pltpu.VMEM((_CPW, _CHUNK), jnp.int32),
            pltpu.VMEM_SHARED((_PE2, _EMBED), jnp.float32),
            pltpu.VMEM((1, _EMBED), jnp.float32),
            pltpu.VMEM((1, _EMBED), jnp.float32),
            pltpu.VMEM((1, _EMBED), jnp.float32),
            pltpu.VMEM((1, _EMBED), jnp.float32),
            pltpu.VMEM((_NB, _CHUNK, _EMBED), jnp.float32),
        ] + [pltpu.SemaphoreType.DMA] * (3 * _NB),
        compiler_params=pltpu.CompilerParams(use_tc_tiling_on_sc=False,
                                             needs_layout_passes=False),
    )(seq, sidx, token_0, token_mask, pe_weight, embeddings)
    x = out.reshape(_BATCH, _MAX_LEN, _EMBED)
    return (x, emb_out)


# speedup vs baseline: 1.0086x; 1.0086x over previous
import jax
import jax.numpy as jnp
from jax import lax
from jax.experimental import pallas as pl
from jax.experimental.pallas import tpu as pltpu
from jax.experimental.pallas import tpu_sc as plsc

_EMBED = 128
_MAX_LEN = 200
_NUM_ITEMS = 100000
_BATCH = 1024

_NC, _NS = 2, 16
_NW = _NC * _NS
_ROWS = _BATCH * _MAX_LEN
_CHUNK = 128
_NCHUNK = _ROWS // _CHUNK
_CPW = _NCHUNK // _NW
_NV = _EMBED // 16
_PE2 = _MAX_LEN + _CHUNK
_NB = 5


def _body(ridx_hbm, sidx_hbm, t0_hbm, tm_hbm, pe_hbm, emb_hbm, out_hbm,
          ridx_v, sidx_v, pe2_sh, t0_v, tm_v, e0_v, eN_v, ob,
          gsem0, gsem1, gsem2, gsem3, gsem4,
          osem0, osem1, osem2, osem3, osem4):
    gsems = (gsem0, gsem1, gsem2, gsem3, gsem4)
    osems = (osem0, osem1, osem2, osem3, osem4)
    sid = lax.axis_index("s")
    w = sid * _NC + lax.axis_index("c")
    base = w * _CPW
    pltpu.sync_copy(ridx_hbm.at[pl.ds(base, _CPW)], ridx_v)
    pltpu.sync_copy(sidx_hbm.at[pl.ds(base, _CPW)], sidx_v)
    pltpu.sync_copy(t0_hbm, t0_v)
    pltpu.sync_copy(tm_hbm, tm_v)
    pltpu.sync_copy(emb_hbm.at[pl.ds(0, 1)], e0_v)
    pltpu.sync_copy(emb_hbm.at[pl.ds(_NUM_ITEMS - 1, 1)], eN_v)

    @pl.when(sid == 0)
    def _stage_pe_shared():
        bounce = ob.at[_NB - 1]
        pltpu.sync_copy(pe_hbm.at[pl.ds(0, _CHUNK)], bounce)
        pltpu.sync_copy(bounce, pe2_sh.at[pl.ds(0, _CHUNK)])
        pltpu.sync_copy(bounce, pe2_sh.at[pl.ds(_MAX_LEN, _CHUNK)])
        rest = _MAX_LEN - _CHUNK
        pltpu.sync_copy(pe_hbm.at[pl.ds(_CHUNK, rest)],
                        bounce.at[pl.ds(0, rest)])
        pltpu.sync_copy(bounce.at[pl.ds(0, rest)],
                        pe2_sh.at[pl.ds(_CHUNK, rest)])

    plsc.subcore_barrier()

    d0r = [t0_v[0, pl.ds(16 * j, 16)] - e0_v[0, pl.ds(16 * j, 16)]
           for j in range(_NV)]
    dmr = [tm_v[0, pl.ds(16 * j, 16)] - eN_v[0, pl.ds(16 * j, 16)]
           for j in range(_NV)]
    zf = jnp.zeros((16,), jnp.float32)

    def start_chunk(c, b):
        tc = (c * _CHUNK) % _MAX_LEN
        pltpu.sync_copy(pe2_sh.at[pl.ds(tc, _CHUNK)], ob.at[b])
        pltpu.async_copy(emb_hbm.at[sidx_v.at[c]], ob.at[b], gsems[b],
                         add=True)

    def fixup(c, b):
        obb = ob.at[b]
        m = None
        for k in range(_NV):
            iv = ridx_v[c, pl.ds(16 * k, 16)]
            mk = (iv == 0) | (iv == _NUM_ITEMS + 1)
            m = mk if m is None else (m | mk)
        any_spec = jnp.any(m)

        @pl.when(any_spec)
        def _slow():
            @pl.loop(0, _CHUNK)
            def _row(l):
                iv = plsc.load_gather(
                    ridx_v,
                    [jnp.full((16,), c, jnp.int32), jnp.full((16,), l, jnp.int32)])
                f0 = iv == 0
                fm = iv == _NUM_ITEMS + 1

                @pl.when(jnp.any(f0 | fm))
                def _patch():
                    for j in range(_NV):
                        v = obb[l, pl.ds(16 * j, 16)]
                        v = v + jnp.where(f0, d0r[j], jnp.where(fm, dmr[j], zf))
                        obb[l, pl.ds(16 * j, 16)] = v

    def out_slice(c):
        return out_hbm.at[pl.ds((base + c) * _CHUNK, _CHUNK)]

    for j in range(_NB - 2):
        start_chunk(j, j)

    @pl.loop(0, _CPW, step=_NB)
    def _ring(c0):
        for j in range(_NB):
            c = c0 + j
            pltpu.make_async_copy(emb_hbm.at[sidx_v.at[c]], ob.at[j],
                                  gsems[j]).wait()
            fixup(c, j)
            pltpu.async_copy(ob.at[j], out_slice(c), osems[j])

            d = c + _NB - 2
            bd = (j + _NB - 2) % _NB

            @pl.when(d < _CPW)
            def _issue():
                @pl.when(c >= 2)
                def _wait_prev_out():
                    pltpu.make_async_copy(ob.at[bd], out_slice(c - 2),
                                          osems[bd]).wait()

                start_chunk(d, bd)

    for j in range(_NB):
        c = _CPW - _NB + j
        pltpu.make_async_copy(ob.at[j], out_slice(c), osems[j]).wait()


def kernel(sequence, token_0, token_mask, pe_weight, embeddings):
    emb_out = jnp.copy(embeddings)
    seq = sequence.reshape(_NCHUNK, _CHUNK).astype(jnp.int32)
    sidx = jnp.clip(seq - 1, 0, _NUM_ITEMS - 1)
    mesh = plsc.VectorSubcoreMesh(core_axis_name="c", subcore_axis_name="s",
                                  num_cores=_NC, num_subcores=_NS)
    out = pl.kernel(
        _body,
        out_type=jax.ShapeDtypeStruct((_ROWS, _EMBED), jnp.float32),
        mesh=mesh,
        scratch_types=[
            pltpu.VMEM((_CPW, _CHUNK), jnp.int32),
            pltpu.VMEM((_CPW, _CHUNK), jnp.int32),
            pltpu.VMEM_SHARED((_PE2, _EMBED), jnp.float32),
            pltpu.VMEM((1, _EMBED), jnp.float32),
            pltpu.VMEM((1, _EMBED), jnp.float32),
            pltpu.VMEM((1, _EMBED), jnp.float32),
            pltpu.VMEM((1, _EMBED), jnp.float32),
            pltpu.VMEM((_NB, _CHUNK, _EMBED), jnp.float32),
        ] + [pltpu.SemaphoreType.DMA] * (2 * _NB),
        compiler_params=pltpu.CompilerParams(use_tc_tiling_on_sc=False,
                                             needs_layout_passes=False),
    )(seq, sidx, token_0, token_mask, pe_weight, embeddings)
    x = out.reshape(_BATCH, _MAX_LEN, _EMBED)
    return (x, emb_out)


# speedup vs baseline: 1.0087x; 1.0001x over previous
import jax
import jax.numpy as jnp
from jax import lax
from jax.experimental import pallas as pl
from jax.experimental.pallas import tpu as pltpu
from jax.experimental.pallas import tpu_sc as plsc

_EMBED = 128
_MAX_LEN = 200
_NUM_ITEMS = 100000
_BATCH = 1024

_NC, _NS = 2, 16
_NW = _NC * _NS
_ROWS = _BATCH * _MAX_LEN
_CHUNK = 128
_NCHUNK = _ROWS // _CHUNK
_CPW = _NCHUNK // _NW
_NV = _EMBED // 16
_PE2 = _MAX_LEN + _CHUNK
_NB = 5


def _body(ridx_hbm, sidx_hbm, t0_hbm, tm_hbm, pe_hbm, emb_hbm, out_hbm,
          ridx_v, sidx_v, pe2_sh, t0_v, tm_v, e0_v, eN_v, ob,
          gsem0, gsem1, gsem2, gsem3, gsem4,
          osem0, osem1, osem2, osem3, osem4):
    gsems = (gsem0, gsem1, gsem2, gsem3, gsem4)
    osems = (osem0, osem1, osem2, osem3, osem4)
    sid = lax.axis_index("s")
    w = sid * _NC + lax.axis_index("c")
    base = w * _CPW
    pltpu.sync_copy(ridx_hbm.at[pl.ds(base, _CPW)], ridx_v)
    pltpu.sync_copy(sidx_hbm.at[pl.ds(base, _CPW)], sidx_v)
    pltpu.sync_copy(t0_hbm, t0_v)
    pltpu.sync_copy(tm_hbm, tm_v)
    pltpu.sync_copy(emb_hbm.at[pl.ds(0, 1)], e0_v)
    pltpu.sync_copy(emb_hbm.at[pl.ds(_NUM_ITEMS - 1, 1)], eN_v)

    @pl.when(sid == 0)
    def _stage_pe_shared():
        bounce = ob.at[_NB - 1]
        pltpu.sync_copy(pe_hbm.at[pl.ds(0, _CHUNK)], bounce)
        pltpu.sync_copy(bounce, pe2_sh.at[pl.ds(0, _CHUNK)])
        pltpu.sync_copy(bounce, pe2_sh.at[pl.ds(_MAX_LEN, _CHUNK)])
        rest = _MAX_LEN - _CHUNK
        pltpu.sync_copy(pe_hbm.at[pl.ds(_CHUNK, rest)],
                        bounce.at[pl.ds(0, rest)])
        pltpu.sync_copy(bounce.at[pl.ds(0, rest)],
                        pe2_sh.at[pl.ds(_CHUNK, rest)])

    plsc.subcore_barrier()

    d0r = [t0_v[0, pl.ds(16 * j, 16)] - e0_v[0, pl.ds(16 * j, 16)]
           for j in range(_NV)]
    dmr = [tm_v[0, pl.ds(16 * j, 16)] - eN_v[0, pl.ds(16 * j, 16)]
           for j in range(_NV)]
    zf = jnp.zeros((16,), jnp.float32)

    def start_chunk(c, b):
        tc = (c * _CHUNK) % _MAX_LEN
        pltpu.sync_copy(pe2_sh.at[pl.ds(tc, _CHUNK)], ob.at[b])
        pltpu.async_copy(emb_hbm.at[sidx_v.at[c]], ob.at[b], gsems[b],
                         add=True)

    def fixup(c, b):
        obb = ob.at[b]
        m = None
        for k in range(_NV):
            iv = ridx_v[c, pl.ds(16 * k, 16)]
            mk = (iv == 0) | (iv == _NUM_ITEMS + 1)
            m = mk if m is None else (m | mk)
        any_spec = jnp.any(m)

        @pl.when(any_spec)
        def _slow():
            @pl.loop(0, _CHUNK)
            def _row(l):
                iv = plsc.load_gather(
                    ridx_v,
                    [jnp.full((16,), c, jnp.int32), jnp.full((16,), l, jnp.int32)])
                f0 = iv == 0
                fm = iv == _NUM_ITEMS + 1

                @pl.when(jnp.any(f0 | fm))
                def _patch():
                    for j in range(_NV):
                        v = obb[l, pl.ds(16 * j, 16)]
                        v = v + jnp.where(f0, d0r[j], jnp.where(fm, dmr[j], zf))
                        obb[l, pl.ds(16 * j, 16)] = v

    def out_slice(c):
        return out_hbm.at[pl.ds((base + c) * _CHUNK, _CHUNK)]

    for j in range(_NB - 2):
        start_chunk(j, j)

    @pl.loop(0, _CPW, step=_NB)
    def _ring(c0):
        for j in range(_NB):
            c = c0 + j
            pltpu.make_async_copy(emb_hbm.at[sidx_v.at[c]], ob.at[j],
                                  gsems[j]).wait()
            fixup(c, j)
            pltpu.async_copy(ob.at[j], out_slice(c), osems[j])

            d = c + _NB - 2
            bd = (j + _NB - 2) % _NB

            @pl.when(d < _CPW)
            def _issue():
                @pl.when(c >= 2)
                def _wait_prev_out():
                    pltpu.make_async_copy(ob.at[bd], out_slice(c - 2),
                                          osems[bd]).wait()

                start_chunk(d, bd)

    for j in range(_NB):
        c = _CPW - _NB + j
        pltpu.make_async_copy(ob.at[j], out_slice(c), osems[j]).wait()


def kernel(sequence, token_0, token_mask, pe_weight, embeddings):
    emb_out = jnp.copy(embeddings)
    seq = sequence.reshape(_NCHUNK, _CHUNK).astype(jnp.int32)
    sidx = jnp.clip(seq - 1, 0, _NUM_ITEMS - 1)
    mesh = plsc.VectorSubcoreMesh(core_axis_name="c", subcore_axis_name="s",
                                  num_cores=_NC, num_subcores=_NS)
    out = pl.kernel(
        _body,
        out_type=jax.ShapeDtypeStruct((_ROWS, _EMBED), jnp.float32),
        mesh=mesh,
        scratch_types=[
            pltpu.VMEM((_CPW, _CHUNK), jnp.int32),
            pltpu.VMEM((_CPW, _CHUNK), jnp.int32),
            pltpu.VMEM_SHARED((_PE2, _EMBED), jnp.float32),
            pltpu.VMEM((1, _EMBED), jnp.float32),
            pltpu.VMEM((1, _EMBED), jnp.float32),
            pltpu.VMEM((1, _EMBED), jnp.float32),
            pltpu.VMEM((1, _EMBED), jnp.float32),
            pltpu.VMEM((_NB, _CHUNK, _EMBED), jnp.float32),
        ] + [pltpu.SemaphoreType.DMA] * (2 * _NB),
        compiler_params=pltpu.CompilerParams(use_tc_tiling_on_sc=False,
                                             needs_layout_passes=False,
                                             skip_device_barrier=True),
    )(seq, sidx, token_0, token_mask, pe_weight, embeddings)
    x = out.reshape(_BATCH, _MAX_LEN, _EMBED)
    return (x, emb_out)


# speedup vs baseline: 1.0102x; 1.0015x over previous
import jax
import jax.numpy as jnp
from jax import lax
from jax.experimental import pallas as pl
from jax.experimental.pallas import tpu as pltpu
from jax.experimental.pallas import tpu_sc as plsc

_EMBED = 128
_MAX_LEN = 200
_NUM_ITEMS = 100000
_BATCH = 1024

_NC, _NS = 2, 16
_NW = _NC * _NS
_ROWS = _BATCH * _MAX_LEN
_CHUNK = 128
_NCHUNK = _ROWS // _CHUNK
_CPW = _NCHUNK // _NW
_NV = _EMBED // 16
_PE2 = _MAX_LEN + _CHUNK
_NB = 5


def _body(ridx_hbm, sidx_hbm, t0_hbm, tm_hbm, pe_hbm, emb_hbm, out_hbm,
          ridx_v, sidx_v, pe2_sh, t0_v, tm_v, e0_v, eN_v, ob,
          gsem0, gsem1, gsem2, gsem3, gsem4,
          osem0, osem1, osem2, osem3, osem4):
    gsems = (gsem0, gsem1, gsem2, gsem3, gsem4)
    osems = (osem0, osem1, osem2, osem3, osem4)
    sid = lax.axis_index("s")
    w = sid * _NC + lax.axis_index("c")
    base = w * _CPW
    pltpu.sync_copy(ridx_hbm.at[pl.ds(base, _CPW)], ridx_v)
    pltpu.sync_copy(sidx_hbm.at[pl.ds(base, _CPW)], sidx_v)
    pltpu.sync_copy(t0_hbm, t0_v)
    pltpu.sync_copy(tm_hbm, tm_v)
    pltpu.sync_copy(emb_hbm.at[pl.ds(0, 1)], e0_v)
    pltpu.sync_copy(emb_hbm.at[pl.ds(_NUM_ITEMS - 1, 1)], eN_v)

    @pl.when(sid == 0)
    def _stage_pe_shared():
        bounce = ob.at[_NB - 1]
        pltpu.sync_copy(pe_hbm.at[pl.ds(0, _CHUNK)], bounce)
        pltpu.sync_copy(bounce, pe2_sh.at[pl.ds(0, _CHUNK)])
        pltpu.sync_copy(bounce, pe2_sh.at[pl.ds(_MAX_LEN, _CHUNK)])
        rest = _MAX_LEN - _CHUNK
        pltpu.sync_copy(pe_hbm.at[pl.ds(_CHUNK, rest)],
                        bounce.at[pl.ds(0, rest)])
        pltpu.sync_copy(bounce.at[pl.ds(0, rest)],
                        pe2_sh.at[pl.ds(_CHUNK, rest)])

    plsc.subcore_barrier()

    d0r = [t0_v[0, pl.ds(16 * j, 16)] - e0_v[0, pl.ds(16 * j, 16)]
           for j in range(_NV)]
    dmr = [tm_v[0, pl.ds(16 * j, 16)] - eN_v[0, pl.ds(16 * j, 16)]
           for j in range(_NV)]
    zf = jnp.zeros((16,), jnp.float32)

    def start_chunk(c, b):
        tc = (c * _CHUNK) % _MAX_LEN
        pltpu.sync_copy(pe2_sh.at[pl.ds(tc, _CHUNK)], ob.at[b])
        pltpu.async_copy(emb_hbm.at[sidx_v.at[c]], ob.at[b], gsems[b],
                         add=True)

    def fixup(c, b):
        obb = ob.at[b]
        m = None
        for k in range(_NV):
            iv = ridx_v[c, pl.ds(16 * k, 16)]
            mk = (iv == 0) | (iv == _NUM_ITEMS + 1)
            m = mk if m is None else (m | mk)
        any_spec = jnp.any(m)

        @pl.when(any_spec)
        def _slow():
            @pl.loop(0, _CHUNK)
            def _row(l):
                iv = plsc.load_gather(
                    ridx_v,
                    [jnp.full((16,), c, jnp.int32), jnp.full((16,), l, jnp.int32)])
                f0 = iv == 0
                fm = iv == _NUM_ITEMS + 1

                @pl.when(jnp.any(f0 | fm))
                def _patch():
                    for j in range(_NV):
                        v = obb[l, pl.ds(16 * j, 16)]
                        v = v + jnp.where(f0, d0r[j], jnp.where(fm, dmr[j], zf))
                        obb[l, pl.ds(16 * j, 16)] = v

    def out_slice(c):
        return out_hbm.at[pl.ds((base + c) * _CHUNK, _CHUNK)]

    for j in range(_NB - 2):
        start_chunk(j, j)

    @pl.loop(0, _CPW, step=_NB)
    def _ring(c0):
        for j in range(_NB):
            c = c0 + j
            pltpu.make_async_copy(emb_hbm.at[sidx_v.at[c]], ob.at[j],
                                  gsems[j]).wait()
            fixup(c, j)
            pltpu.async_copy(ob.at[j], out_slice(c), osems[j])

            d = c + _NB - 2
            bd = (j + _NB - 2) % _NB

            @pl.when(d < _CPW)
            def _issue():
                @pl.when(c >= 2)
                def _wait_prev_out():
                    pltpu.make_async_copy(ob.at[bd], out_slice(c - 2),
                                          osems[bd]).wait()

                start_chunk(d, bd)

    for j in range(_NB):
        c = _CPW - _NB + j
        pltpu.make_async_copy(ob.at[j], out_slice(c), osems[j]).wait()


def kernel(sequence, token_0, token_mask, pe_weight, embeddings):
    emb_out = jnp.copy(embeddings)
    seq = sequence.reshape(_NCHUNK, _CHUNK).astype(jnp.int32)
    sidx = jnp.clip(seq - 1, 0, _NUM_ITEMS - 1)
    mesh = plsc.VectorSubcoreMesh(core_axis_name="c", subcore_axis_name="s",
                                  num_cores=_NC, num_subcores=_NS)
    out = pl.kernel(
        _body,
        out_type=jax.ShapeDtypeStruct((_ROWS, _EMBED), jnp.float32),
        mesh=mesh,
        scratch_types=[
            pltpu.VMEM((_CPW, _CHUNK), jnp.int32),
            pltpu.VMEM((_CPW, _CHUNK), jnp.int32),
            pltpu.VMEM_SHARED((_PE2, _EMBED), jnp.float32),
            pltpu.VMEM((1, _EMBED), jnp.float32),
            pltpu.VMEM((1, _EMBED), jnp.float32),
            pltpu.VMEM((1, _EMBED), jnp.float32),
            pltpu.VMEM((1, _EMBED), jnp.float32),
            pltpu.VMEM((_NB, _CHUNK, _EMBED), jnp.float32),
        ] + [pltpu.SemaphoreType.DMA] * (2 * _NB),
        compiler_params=pltpu.CompilerParams(use_tc_tiling_on_sc=False,
                                             needs_layout_passes=False),
    )(seq, sidx, token_0, token_mask, pe_weight, embeddings)
    x = out.reshape(_BATCH, _MAX_LEN, _EMBED)
    return (x, emb_out)
